# Initial kernel scaffold; baseline (speedup 1.0000x reference)
import jax
import jax.numpy as jnp
from jax.experimental import pallas as pl


def kernel(x, edge_index, edge_attr, batch, params):
    raise NotImplementedError("write your pallas kernel here")


# speedup vs baseline: 1.1325x; 1.1325x over previous
import functools

import jax
import jax.numpy as jnp
from jax.experimental import pallas as pl
from jax.experimental.pallas import tpu as pltpu

HEADS = 8
HC = 32


def _dense_ln_relu_block(x_ref, w_ref, b_ref, g_ref, beta_ref, o_ref):
    h = jnp.dot(x_ref[...], w_ref[...], preferred_element_type=jnp.float32)
    h = h + b_ref[...]
    m = jnp.mean(h, axis=-1, keepdims=True)
    v = jnp.mean((h - m) ** 2, axis=-1, keepdims=True)
    h = (h - m) * jax.lax.rsqrt(v + 1e-5) * g_ref[...] + beta_ref[...]
    o_ref[...] = jnp.maximum(h, 0.0)


def _dense_ln_relu(x, W, b, g, beta, bn=1000):
    N, K = x.shape
    H = W.shape[1]
    b2 = b.reshape(1, H)
    g2 = g.reshape(1, H)
    beta2 = beta.reshape(1, H)
    return pl.pallas_call(
        _dense_ln_relu_block,
        grid=(N // bn,),
        in_specs=[
            pl.BlockSpec((bn, K), lambda i: (i, 0)),
            pl.BlockSpec((K, H), lambda i: (0, 0)),
            pl.BlockSpec((1, H), lambda i: (0, 0)),
            pl.BlockSpec((1, H), lambda i: (0, 0)),
            pl.BlockSpec((1, H), lambda i: (0, 0)),
        ],
        out_specs=pl.BlockSpec((bn, H), lambda i: (i, 0)),
        out_shape=jax.ShapeDtypeStruct((N, H), jnp.float32),
    )(x, W, b2, g2, beta2)


def _ln(v, g, b, eps=1e-5):
    m = jnp.mean(v, axis=-1, keepdims=True)
    var = jnp.var(v, axis=-1, keepdims=True)
    return (v - m) / jnp.sqrt(var + eps) * g + b


def _leaky(x):
    return jnp.where(x >= 0, x, 0.2 * x)


def kernel(x, edge_index, edge_attr, batch, params):
    n = x.shape[0]
    src = edge_index[0]
    dst = edge_index[1]

    h = _dense_ln_relu(x, params["in_W"], params["in_b"], params["in_g"],
                       params["in_beta"])

    deg = 1.0 + jax.ops.segment_sum(jnp.ones_like(src, jnp.float32), dst,
                                    num_segments=n)
    dinv = deg ** -0.5
    norm = dinv[src] * dinv[dst]
    mean_ea = jnp.mean(edge_attr, axis=0)

    for i in range(5):
        p = params["layers"][i]
        W = p["W"]
        if i % 2 == 0:
            xl = h @ W
            A_src = jnp.einsum("dk,k->dk" if False else "dhc,hc->dh",
                               W.reshape(-1, HEADS, HC), p["att_src"][0])
            A_dst = jnp.einsum("dhc,hc->dh", W.reshape(-1, HEADS, HC),
                               p["att_dst"][0])
            Ve = jnp.einsum("dhc,hc->dh", p["W_e"].reshape(-1, HEADS, HC),
                            p["att_e"][0])
            a_s = h @ A_src
            a_d = h @ A_dst
            a_e = edge_attr @ Ve
            a_e_loop = mean_ea @ Ve

            pe = jnp.exp(_leaky(a_s[src] + a_d[dst] + a_e))
            p_self = jnp.exp(_leaky(a_s + a_d + a_e_loop))

            xl3 = xl.reshape(n, HEADS, HC)
            denom = jax.ops.segment_sum(pe, dst, num_segments=n) + p_self
            num = jax.ops.segment_sum(xl3[src] * pe[:, :, None], dst,
                                      num_segments=n)
            num = num + xl3 * p_self[:, :, None]
            out = (num / (denom[:, :, None] + 1e-16)).reshape(n, HEADS * HC)
            h = out + p["b"]
        else:
            xw = h @ W
            out = jax.ops.segment_sum(xw[src] * norm[:, None], dst,
                                      num_segments=n)
            h = out + xw * (dinv * dinv)[:, None] + p["b"]
        h = jnp.maximum(_ln(h, params["ln_g"][i], params["ln_b"][i]), 0.0)

    add_p = jnp.sum(h, axis=0, keepdims=True)
    mean_p = add_p / n
    max_p = jnp.max(h, axis=0, keepdims=True)
    pooled = jnp.concatenate([mean_p, max_p, add_p], axis=1)

    z = pooled @ params["m0_W"] + params["m0_b"]
    z = jax.nn.relu(_ln(z, params["m0_g"], params["m0_beta"]))
    z = z @ params["m1_W"] + params["m1_b"]
    z = jax.nn.relu(_ln(z, params["m1_g"], params["m1_beta"]))
    return z @ params["m2_W"] + params["m2_b"]


# speedup vs baseline: 19.3563x; 17.0924x over previous
import functools

import jax
import jax.numpy as jnp
from jax import lax
from jax.experimental import pallas as pl
from jax.experimental.pallas import tpu as pltpu
from jax.experimental.pallas import tpu_sc as plsc

N = 10000
E = 320000
HEADS = 8
HC = 32
HID = 256

ROWW = 144
KA = 160
KC = 160
NT = 16
NP = 10240
NROWS = NP // NT
EW = E // NT
EW32 = E // 32


def _ln_relu(h, g, beta):
    m = jnp.mean(h, axis=-1, keepdims=True)
    v = jnp.mean((h - m) ** 2, axis=-1, keepdims=True)
    return jnp.maximum((h - m) / jnp.sqrt(v + 1e-5) * g + beta, 0.0)


def _in_dense_block(x_ref, w_ref, b_ref, g_ref, beta_ref, o_ref):
    h = jnp.dot(x_ref[...], w_ref[...], preferred_element_type=jnp.float32)
    o_ref[...] = _ln_relu(h + b_ref[...], g_ref[...], beta_ref[...])


def _in_dense(x, W, b, g, beta, bn=1000):
    n, k = x.shape
    hd = W.shape[1]
    return pl.pallas_call(
        _in_dense_block,
        grid=(n // bn,),
        in_specs=[
            pl.BlockSpec((bn, k), lambda i: (i, 0)),
            pl.BlockSpec((k, hd), lambda i: (0, 0)),
            pl.BlockSpec((1, hd), lambda i: (0, 0)),
            pl.BlockSpec((1, hd), lambda i: (0, 0)),
            pl.BlockSpec((1, hd), lambda i: (0, 0)),
        ],
        out_specs=pl.BlockSpec((bn, hd), lambda i: (i, 0)),
        out_shape=jax.ShapeDtypeStruct((n, hd), jnp.float32),
    )(x, W, b.reshape(1, -1), g.reshape(1, -1), beta.reshape(1, -1))


def _gat_mm_block(h_ref, w_ref, as_att_ref, ad_att_ref, xl_ref, as_ref,
                  ad_ref):
    xl = jnp.dot(h_ref[...], w_ref[...], preferred_element_type=jnp.float32)
    xl_ref[...] = xl
    xl3 = xl.reshape(xl.shape[0], HEADS, HC)
    as_ref[...] = jnp.sum(xl3 * as_att_ref[...], axis=-1)
    ad_ref[...] = jnp.sum(xl3 * ad_att_ref[...], axis=-1)


def _gat_mm(h, W, att_s, att_d, bn=1000):
    n = h.shape[0]
    return pl.pallas_call(
        _gat_mm_block,
        grid=(n // bn,),
        in_specs=[
            pl.BlockSpec((bn, HID), lambda i: (i, 0)),
            pl.BlockSpec((HID, HID), lambda i: (0, 0)),
            pl.BlockSpec((1, HEADS, HC), lambda i: (0, 0, 0)),
            pl.BlockSpec((1, HEADS, HC), lambda i: (0, 0, 0)),
        ],
        out_specs=[
            pl.BlockSpec((bn, HID), lambda i: (i, 0)),
            pl.BlockSpec((bn, HEADS), lambda i: (i, 0)),
            pl.BlockSpec((bn, HEADS), lambda i: (i, 0)),
        ],
        out_shape=[
            jax.ShapeDtypeStruct((n, HID), jnp.float32),
            jax.ShapeDtypeStruct((n, HEADS), jnp.float32),
            jax.ShapeDtypeStruct((n, HEADS), jnp.float32),
        ],
    )(h, W, att_s, att_d)


def _mm_block(h_ref, w_ref, o_ref):
    o_ref[...] = jnp.dot(h_ref[...], w_ref[...],
                         preferred_element_type=jnp.float32)


def _plain_mm(h, W, bn=1000):
    n, k = h.shape
    hd = W.shape[1]
    return pl.pallas_call(
        _mm_block,
        grid=(n // bn,),
        in_specs=[
            pl.BlockSpec((bn, k), lambda i: (i, 0)),
            pl.BlockSpec((k, hd), lambda i: (0, 0)),
        ],
        out_specs=pl.BlockSpec((bn, hd), lambda i: (i, 0)),
        out_shape=jax.ShapeDtypeStruct((n, hd), jnp.float32),
    )(h, W)


def _gat_post_block(num_ref, den_ref, b_ref, g_ref, beta_ref, o_ref):
    num = num_ref[...]
    bn = num.shape[0]
    num3 = num.reshape(bn, HEADS, HC)
    den3 = den_ref[...].reshape(bn, HEADS, 1)
    h = (num3 / (den3 + 1e-16)).reshape(bn, HID) + b_ref[...]
    o_ref[...] = _ln_relu(h, g_ref[...], beta_ref[...])


def _gat_post(num, den, b, g, beta, bn=1000):
    n = num.shape[0]
    return pl.pallas_call(
        _gat_post_block,
        grid=(n // bn,),
        in_specs=[
            pl.BlockSpec((bn, HID), lambda i: (i, 0)),
            pl.BlockSpec((bn, HEADS), lambda i: (i, 0)),
            pl.BlockSpec((1, HID), lambda i: (0, 0)),
            pl.BlockSpec((1, HID), lambda i: (0, 0)),
            pl.BlockSpec((1, HID), lambda i: (0, 0)),
        ],
        out_specs=pl.BlockSpec((bn, HID), lambda i: (i, 0)),
        out_shape=jax.ShapeDtypeStruct((n, HID), jnp.float32),
    )(num, den, b.reshape(1, -1), g.reshape(1, -1), beta.reshape(1, -1))


def _gcn_post_block(agg_ref, b_ref, g_ref, beta_ref, o_ref):
    o_ref[...] = _ln_relu(agg_ref[...] + b_ref[...], g_ref[...], beta_ref[...])


def _gcn_post(agg, b, g, beta, bn=1000):
    n = agg.shape[0]
    return pl.pallas_call(
        _gcn_post_block,
        grid=(n // bn,),
        in_specs=[
            pl.BlockSpec((bn, HID), lambda i: (i, 0)),
            pl.BlockSpec((1, HID), lambda i: (0, 0)),
            pl.BlockSpec((1, HID), lambda i: (0, 0)),
            pl.BlockSpec((1, HID), lambda i: (0, 0)),
        ],
        out_specs=pl.BlockSpec((bn, HID), lambda i: (i, 0)),
        out_shape=jax.ShapeDtypeStruct((n, HID), jnp.float32),
    )(agg, b.reshape(1, -1), g.reshape(1, -1), beta.reshape(1, -1))


def _edge_scores_block(ea_ref, we_ref, att_ref, o_ref):
    ew = jnp.dot(ea_ref[...], we_ref[...], preferred_element_type=jnp.float32)
    e3 = ew.reshape(ew.shape[0], HEADS, HC)
    o_ref[...] = jnp.sum(e3 * att_ref[...], axis=-1)


def _edge_scores(ea, We, att_e, bn=8000):
    e = ea.shape[0]
    return pl.pallas_call(
        _edge_scores_block,
        grid=(e // bn,),
        in_specs=[
            pl.BlockSpec((bn, 16), lambda i: (i, 0)),
            pl.BlockSpec((16, HID), lambda i: (0, 0)),
            pl.BlockSpec((1, HEADS, HC), lambda i: (0, 0, 0)),
        ],
        out_specs=pl.BlockSpec((bn, HEADS), lambda i: (i, 0)),
        out_shape=jax.ShapeDtypeStruct((e, HEADS), jnp.float32),
    )(ea, We, att_e)


def _pool_head_block(h_ref, w0, b0, g0, be0, w1, b1, g1, be1, w2, b2,
                     o_ref, ssum, smax):
    i = pl.program_id(0)
    blk = h_ref[...]
    bs = jnp.sum(blk, axis=0, keepdims=True)
    bm = jnp.max(blk, axis=0, keepdims=True)

    @pl.when(i == 0)
    def _():
        ssum[...] = bs
        smax[...] = bm

    @pl.when(i > 0)
    def _():
        ssum[...] = ssum[...] + bs
        smax[...] = jnp.maximum(smax[...], bm)

    @pl.when(i == pl.num_programs(0) - 1)
    def _():
        add_p = ssum[...]
        pooled = jnp.concatenate([add_p / N, smax[...], add_p], axis=1)
        hi = lax.Precision.HIGHEST
        z = jnp.dot(pooled, w0[...], preferred_element_type=jnp.float32,
                    precision=hi)
        z = z + b0[...]
        m = jnp.mean(z, axis=-1, keepdims=True)
        v = jnp.mean((z - m) ** 2, axis=-1, keepdims=True)
        z = jnp.maximum((z - m) * lax.rsqrt(v + 1e-5) * g0[...] + be0[...], 0.)
        z = jnp.dot(z, w1[...], preferred_element_type=jnp.float32,
                    precision=hi) + b1[...]
        m = jnp.mean(z, axis=-1, keepdims=True)
        v = jnp.mean((z - m) ** 2, axis=-1, keepdims=True)
        z = jnp.maximum((z - m) * lax.rsqrt(v + 1e-5) * g1[...] + be1[...], 0.)
        o_ref[...] = jnp.dot(z, w2[...], preferred_element_type=jnp.float32,
                             precision=hi) + b2[...]


def _pool_head(h, p, bn=1000):
    n = h.shape[0]
    c = lambda i: (0, 0)
    return pl.pallas_call(
        _pool_head_block,
        grid=(n // bn,),
        in_specs=[
            pl.BlockSpec((bn, HID), lambda i: (i, 0)),
            pl.BlockSpec((3 * HID, HID), c),
            pl.BlockSpec((1, HID), c),
            pl.BlockSpec((1, HID), c),
            pl.BlockSpec((1, HID), c),
            pl.BlockSpec((HID, HID), c),
            pl.BlockSpec((1, HID), c),
            pl.BlockSpec((1, HID), c),
            pl.BlockSpec((1, HID), c),
            pl.BlockSpec((HID, 1), c),
            pl.BlockSpec((1, 1), c),
        ],
        out_specs=pl.BlockSpec((1, 1), c),
        out_shape=jax.ShapeDtypeStruct((1, 1), jnp.float32),
        scratch_shapes=[
            pltpu.VMEM((1, HID), jnp.float32),
            pltpu.VMEM((1, HID), jnp.float32),
        ],
    )(h, p["m0_W"], p["m0_b"].reshape(1, -1), p["m0_g"].reshape(1, -1),
      p["m0_beta"].reshape(1, -1), p["m1_W"], p["m1_b"].reshape(1, -1),
      p["m1_g"].reshape(1, -1), p["m1_beta"].reshape(1, -1), p["m2_W"],
      p["m2_b"].reshape(1, -1))


_MESH = dict(core_axis_name="c", subcore_axis_name="s")


def _sc_deg_body(d_hbm, zinit, out, acc, dbuf, stage):
    c = lax.axis_index("c")
    t = lax.axis_index("s")
    w = t * 2 + c
    pltpu.sync_copy(zinit.at[c, pl.ds(t * NROWS, NROWS)],
                    acc.at[pl.ds(t * NROWS, NROWS)])
    lanes = jnp.arange(16, dtype=jnp.int32)
    onev = jnp.where(lanes == 0, 1.0, 0.0).astype(jnp.float32)

    def fill(r, carry):
        stage[r, pl.ds(0, 16)] = onev
        return carry
    lax.fori_loop(0, 2000, fill, None)
    plsc.subcore_barrier()

    def chunk(i, carry):
        b = w * EW32 + i * 2000
        pltpu.sync_copy(d_hbm.at[pl.ds(b, 2000)], dbuf)
        pltpu.sync_copy(stage, acc.at[dbuf], add=True)
        return carry
    lax.fori_loop(0, EW32 // 2000, chunk, None)
    plsc.subcore_barrier()
    pltpu.sync_copy(acc.at[pl.ds(t * NROWS, NROWS)],
                    out.at[c, pl.ds(t * NROWS, NROWS)])


def _sc_deg(d):
    zinit = jnp.zeros((2, NP, 16), jnp.float32)
    k = pl.kernel(
        _sc_deg_body,
        out_type=jax.ShapeDtypeStruct((2, NP, 16), jnp.float32),
        mesh=plsc.VectorSubcoreMesh(**_MESH),
        compiler_params=pltpu.CompilerParams(use_tc_tiling_on_sc=False, needs_layout_passes=False),
        scratch_types=[
            pltpu.VMEM_SHARED((NP, 16), jnp.float32),
            pltpu.VMEM((2000,), jnp.int32),
            pltpu.VMEM((2000, 16), jnp.float32),
        ],
    )
    return k(d, zinit)


def _sc_norm_body(s_hbm, d_hbm, dinv_hbm, out, dt, sbuf, dbuf, nbuf):
    c = lax.axis_index("c")
    t = lax.axis_index("s")
    w = t * 2 + c
    pltpu.sync_copy(dinv_hbm, dt)
    lanes = jnp.arange(16, dtype=jnp.int32)

    def chunk(i, carry):
        b = w * EW32 + i * 2000
        pltpu.sync_copy(s_hbm.at[pl.ds(b, 2000)], sbuf)
        pltpu.sync_copy(d_hbm.at[pl.ds(b, 2000)], dbuf)

        def grp(g, carry2):
            s16 = sbuf[pl.ds(g * 16, 16)]
            d16 = dbuf[pl.ds(g * 16, 16)]
            nv = plsc.load_gather(dt, [s16]) * plsc.load_gather(dt, [d16])
            nbuf[pl.ds(g * 16, 16)] = nv
            return carry2
        lax.fori_loop(0, 125, grp, None)
        pltpu.sync_copy(nbuf, out.at[pl.ds(b, 2000)])
        return carry
    lax.fori_loop(0, EW32 // 2000, chunk, None)


def _sc_norm(s, d, dinv):
    k = pl.kernel(
        _sc_norm_body,
        out_type=jax.ShapeDtypeStruct((E,), jnp.float32),
        mesh=plsc.VectorSubcoreMesh(**_MESH),
        compiler_params=pltpu.CompilerParams(use_tc_tiling_on_sc=False, needs_layout_passes=False),
        scratch_types=[
            pltpu.VMEM((N,), jnp.float32),
            pltpu.VMEM((2000,), jnp.int32),
            pltpu.VMEM((2000,), jnp.int32),
            pltpu.VMEM((2000,), jnp.float32),
        ],
    )
    return k(s, d, dinv)


def _sc_gat_body(xt, sth, dth, aeh, s2, d2, d_hbm, inith, out,
                 acc, xbuf, sgbuf, dgbuf, sbuf, d2buf, dbuf, aebuf, pbuf,
                 sem):
    c = lax.axis_index("c")
    t = lax.axis_index("s")
    pltpu.sync_copy(inith.at[c, pl.ds(t * NROWS, NROWS)],
                    acc.at[pl.ds(t * NROWS, NROWS)])
    pbuf[pl.ds(4 * KA, 16)] = jnp.zeros((16,), jnp.float32)
    plsc.subcore_barrier()
    lanes = jnp.arange(16, dtype=jnp.int32)
    base = t * EW

    def chunk(i, carry):
        b = base + i * KA
        cb = c * E + b
        pltpu.sync_copy(s2.at[pl.ds(cb, KA)], sbuf)
        pltpu.sync_copy(d2.at[pl.ds(cb, KA)], d2buf)
        pltpu.sync_copy(d_hbm.at[pl.ds(b, KA)], dbuf)
        for h in range(4):
            pltpu.sync_copy(aeh.at[pl.ds((c * 4 + h) * E + b, KA)],
                            aebuf.at[pl.ds(h * KA, KA)])
        g1 = pltpu.async_copy(xt.at[sbuf], xbuf, sem)
        g2 = pltpu.async_copy(sth.at[sbuf], sgbuf, sem)
        g3 = pltpu.async_copy(dth.at[d2buf], dgbuf, sem)
        g1.wait()
        g2.wait()
        g3.wait()

        def score(g, carry2):
            lid = g * 16 + lanes
            for h in range(4):
                hv = jnp.full((16,), h, jnp.int32)
                a_s = plsc.load_gather(sgbuf, [lid, hv])
                a_d = plsc.load_gather(dgbuf, [lid, hv])
                raw = a_s + a_d + aebuf[pl.ds(h * KA + g * 16, 16)]
                raw = jnp.where(raw >= 0, raw, 0.2 * raw)
                pbuf[pl.ds(h * KA + g * 16, 16)] = jnp.exp(raw)
            return carry2
        lax.fori_loop(0, KA // 16, score, None)

        def msg(e, carry2):
            pv = plsc.load_gather(
                pbuf, [jnp.where(lanes < 4, lanes * KA + e, 4 * KA)])
            for h in range(4):
                wv = plsc.load_gather(
                    pbuf, [jnp.full((16,), h * KA, jnp.int32) + e])
                for q in range(2):
                    col = h * 32 + q * 16
                    xbuf[e, pl.ds(col, 16)] = xbuf[e, pl.ds(col, 16)] * wv
            xbuf[e, pl.ds(128, 16)] = pv
            return carry2
        lax.fori_loop(0, KA, msg, None)
        pltpu.sync_copy(xbuf, acc.at[dbuf], add=True)
        return carry
    lax.fori_loop(0, EW // KA, chunk, None)
    plsc.subcore_barrier()
    pltpu.sync_copy(acc.at[pl.ds(t * NROWS, NROWS)],
                    out.at[c, pl.ds(t * NROWS, NROWS)])


def _sc_gat(xt, sth, dth, aeh, s2, d2, d, inith):
    k = pl.kernel(
        _sc_gat_body,
        out_type=jax.ShapeDtypeStruct((2, NP, ROWW), jnp.float32),
        mesh=plsc.VectorSubcoreMesh(**_MESH),
        compiler_params=pltpu.CompilerParams(use_tc_tiling_on_sc=False, needs_layout_passes=False),
        scratch_types=[
            pltpu.VMEM_SHARED((NP, ROWW), jnp.float32),
            pltpu.VMEM((KA, ROWW), jnp.float32),
            pltpu.VMEM((KA, 8), jnp.float32),
            pltpu.VMEM((KA, 8), jnp.float32),
            pltpu.VMEM((KA,), jnp.int32),
            pltpu.VMEM((KA,), jnp.int32),
            pltpu.VMEM((KA,), jnp.int32),
            pltpu.VMEM((4 * KA,), jnp.float32),
            pltpu.VMEM((4 * KA + 16,), jnp.float32),
            pltpu.SemaphoreType.DMA,
        ],
    )
    return k(xt, sth, dth, aeh, s2, d2, d, inith)


def _sc_gcn_body(xt, s2, d_hbm, nrm, inith, out,
                 acc, xbuf, sbuf, dbuf, nbuf, sem):
    c = lax.axis_index("c")
    t = lax.axis_index("s")
    pltpu.sync_copy(inith.at[c, pl.ds(t * NROWS, NROWS)],
                    acc.at[pl.ds(t * NROWS, NROWS)])
    plsc.subcore_barrier()
    base = t * EW

    def chunk(i, carry):
        b = base + i * KC
        pltpu.sync_copy(s2.at[pl.ds(c * E + b, KC)], sbuf)
        pltpu.sync_copy(d_hbm.at[pl.ds(b, KC)], dbuf)
        pltpu.sync_copy(nrm.at[pl.ds(b, KC)], nbuf)
        pltpu.async_copy(xt.at[sbuf], xbuf, sem).wait()

        def msg(e, carry2):
            wv = plsc.load_gather(nbuf, [jnp.full((16,), 0, jnp.int32) + e])
            for q in range(8):
                xbuf[e, pl.ds(q * 16, 16)] = xbuf[e, pl.ds(q * 16, 16)] * wv
            return carry2
        lax.fori_loop(0, KC, msg, None)
        pltpu.sync_copy(xbuf, acc.at[dbuf], add=True)
        return carry
    lax.fori_loop(0, EW // KC, chunk, None)
    plsc.subcore_barrier()
    pltpu.sync_copy(acc.at[pl.ds(t * NROWS, NROWS)],
                    out.at[c, pl.ds(t * NROWS, NROWS)])


def _sc_gcn(xt, s2, d, nrm, inith):
    k = pl.kernel(
        _sc_gcn_body,
        out_type=jax.ShapeDtypeStruct((2, NP, 128), jnp.float32),
        mesh=plsc.VectorSubcoreMesh(**_MESH),
        compiler_params=pltpu.CompilerParams(use_tc_tiling_on_sc=False, needs_layout_passes=False),
        scratch_types=[
            pltpu.VMEM_SHARED((NP, 128), jnp.float32),
            pltpu.VMEM((KC, 128), jnp.float32),
            pltpu.VMEM((KC,), jnp.int32),
            pltpu.VMEM((KC,), jnp.int32),
            pltpu.VMEM((KC,), jnp.float32),
            pltpu.SemaphoreType.DMA,
        ],
    )
    return k(xt, s2, d, nrm, inith)


def _padn(a):
    return jnp.concatenate(
        [a, jnp.zeros((2, NP - N, a.shape[2]), jnp.float32)], axis=1)


def kernel(x, edge_index, edge_attr, batch, params):
    f32 = jnp.float32
    src = edge_index[0]
    dst = edge_index[1]
    s2 = jnp.concatenate([src, src + N])
    d2 = jnp.concatenate([dst, dst + N])

    h = _in_dense(x, params["in_W"], params["in_b"], params["in_g"],
                  params["in_beta"])

    degp = _sc_deg(dst)
    deg = 1.0 + degp[0, :N, 0] + degp[1, :N, 0]
    dinv = deg ** -0.5
    nrm = _sc_norm(src, dst, dinv)
    dinv2 = dinv * dinv

    mean_ea = jnp.mean(edge_attr, axis=0)

    for i in range(5):
        p = params["layers"][i]
        if i % 2 == 0:
            xl, a_s, a_d = _gat_mm(h, p["W"], p["att_src"], p["att_dst"])
            ae = _edge_scores(edge_attr, p["W_e"], p["att_e"])
            ae_loop = jnp.sum(
                (mean_ea @ p["W_e"]).reshape(HEADS, HC) * p["att_e"][0],
                axis=-1)

            p_self = jnp.exp(jnp.where(
                a_s + a_d + ae_loop[None] >= 0,
                a_s + a_d + ae_loop[None],
                0.2 * (a_s + a_d + ae_loop[None])))

            xl_sp = xl.reshape(N, 2, 128).transpose(1, 0, 2)
            xt = jnp.concatenate(
                [xl_sp, jnp.zeros((2, N, 16), f32)], axis=2
            ).reshape(2 * N, ROWW)
            z4 = jnp.zeros((2, N, 4), f32)
            sth = jnp.concatenate(
                [a_s.reshape(N, 2, 4).transpose(1, 0, 2), z4],
                axis=2).reshape(2 * N, 8)
            dth = jnp.concatenate(
                [a_d.reshape(N, 2, 4).transpose(1, 0, 2), z4],
                axis=2).reshape(2 * N, 8)
            aeh = ae.T.reshape(-1)
            ps_sp = p_self.reshape(N, 2, 4).transpose(1, 0, 2)
            inith = jnp.concatenate(
                [xl_sp * jnp.repeat(ps_sp, HC, axis=2),
                 ps_sp, jnp.zeros((2, N, 12), f32)], axis=2)

            agg = _sc_gat(xt, sth, dth, aeh, s2, d2, dst, _padn(inith))
            num = jnp.concatenate([agg[0, :N, :128], agg[1, :N, :128]], axis=1)
            den = jnp.concatenate(
                [agg[0, :N, 128:132], agg[1, :N, 128:132]], axis=1)
            h = _gat_post(num, den, p["b"], params["ln_g"][i],
                          params["ln_b"][i])
        else:
            xw = _plain_mm(h, p["W"])
            xw_sp = xw.reshape(N, 2, 128).transpose(1, 0, 2)
            inith = xw_sp * dinv2[None, :, None]
            agg = _sc_gcn(xw_sp.reshape(2 * N, 128), s2, dst, nrm,
                          _padn(inith))
            aggf = jnp.concatenate([agg[0, :N], agg[1, :N]], axis=1)
            h = _gcn_post(aggf, p["b"], params["ln_g"][i],
                          params["ln_b"][i])

    return _pool_head(h, params)


# speedup vs baseline: 23.0050x; 1.1885x over previous
import functools

import jax
import jax.numpy as jnp
from jax import lax
from jax.experimental import pallas as pl
from jax.experimental.pallas import tpu as pltpu
from jax.experimental.pallas import tpu_sc as plsc

N = 10000
E = 320000
HEADS = 8
HC = 32
HID = 256

ROWW = 144
KA = 160
KC = 160
NT = 16
NP = 10240
NROWS = NP // NT
EW = E // NT
EW32 = E // 32


def _ln_relu(h, g, beta):
    m = jnp.mean(h, axis=-1, keepdims=True)
    v = jnp.mean((h - m) ** 2, axis=-1, keepdims=True)
    return jnp.maximum((h - m) / jnp.sqrt(v + 1e-5) * g + beta, 0.0)


def _in_dense_block(x_ref, w_ref, b_ref, g_ref, beta_ref, o_ref):
    h = jnp.dot(x_ref[...], w_ref[...], preferred_element_type=jnp.float32)
    o_ref[...] = _ln_relu(h + b_ref[...], g_ref[...], beta_ref[...])


def _in_dense(x, W, b, g, beta, bn=1000):
    n, k = x.shape
    hd = W.shape[1]
    return pl.pallas_call(
        _in_dense_block,
        grid=(n // bn,),
        in_specs=[
            pl.BlockSpec((bn, k), lambda i: (i, 0)),
            pl.BlockSpec((k, hd), lambda i: (0, 0)),
            pl.BlockSpec((1, hd), lambda i: (0, 0)),
            pl.BlockSpec((1, hd), lambda i: (0, 0)),
            pl.BlockSpec((1, hd), lambda i: (0, 0)),
        ],
        out_specs=pl.BlockSpec((bn, hd), lambda i: (i, 0)),
        out_shape=jax.ShapeDtypeStruct((n, hd), jnp.float32),
    )(x, W, b.reshape(1, -1), g.reshape(1, -1), beta.reshape(1, -1))


def _gat_mm_block(h_ref, w_ref, as_att_ref, ad_att_ref, xl_ref, as_ref,
                  ad_ref):
    xl = jnp.dot(h_ref[...], w_ref[...], preferred_element_type=jnp.float32)
    xl_ref[...] = xl
    xl3 = xl.reshape(xl.shape[0], HEADS, HC)
    as_ref[...] = jnp.sum(xl3 * as_att_ref[...], axis=-1)
    ad_ref[...] = jnp.sum(xl3 * ad_att_ref[...], axis=-1)


def _gat_mm(h, W, att_s, att_d, bn=1000):
    n = h.shape[0]
    return pl.pallas_call(
        _gat_mm_block,
        grid=(n // bn,),
        in_specs=[
            pl.BlockSpec((bn, HID), lambda i: (i, 0)),
            pl.BlockSpec((HID, HID), lambda i: (0, 0)),
            pl.BlockSpec((1, HEADS, HC), lambda i: (0, 0, 0)),
            pl.BlockSpec((1, HEADS, HC), lambda i: (0, 0, 0)),
        ],
        out_specs=[
            pl.BlockSpec((bn, HID), lambda i: (i, 0)),
            pl.BlockSpec((bn, HEADS), lambda i: (i, 0)),
            pl.BlockSpec((bn, HEADS), lambda i: (i, 0)),
        ],
        out_shape=[
            jax.ShapeDtypeStruct((n, HID), jnp.float32),
            jax.ShapeDtypeStruct((n, HEADS), jnp.float32),
            jax.ShapeDtypeStruct((n, HEADS), jnp.float32),
        ],
    )(h, W, att_s, att_d)


def _mm_block(h_ref, w_ref, o_ref):
    o_ref[...] = jnp.dot(h_ref[...], w_ref[...],
                         preferred_element_type=jnp.float32)


def _plain_mm(h, W, bn=1000):
    n, k = h.shape
    hd = W.shape[1]
    return pl.pallas_call(
        _mm_block,
        grid=(n // bn,),
        in_specs=[
            pl.BlockSpec((bn, k), lambda i: (i, 0)),
            pl.BlockSpec((k, hd), lambda i: (0, 0)),
        ],
        out_specs=pl.BlockSpec((bn, hd), lambda i: (i, 0)),
        out_shape=jax.ShapeDtypeStruct((n, hd), jnp.float32),
    )(h, W)


def _gat_post_block(num_ref, den_ref, b_ref, g_ref, beta_ref, o_ref):
    num = num_ref[...]
    bn = num.shape[0]
    num3 = num.reshape(bn, HEADS, HC)
    den3 = den_ref[...].reshape(bn, HEADS, 1)
    h = (num3 / (den3 + 1e-16)).reshape(bn, HID) + b_ref[...]
    o_ref[...] = _ln_relu(h, g_ref[...], beta_ref[...])


def _gat_post(num, den, b, g, beta, bn=1000):
    n = num.shape[0]
    return pl.pallas_call(
        _gat_post_block,
        grid=(n // bn,),
        in_specs=[
            pl.BlockSpec((bn, HID), lambda i: (i, 0)),
            pl.BlockSpec((bn, HEADS), lambda i: (i, 0)),
            pl.BlockSpec((1, HID), lambda i: (0, 0)),
            pl.BlockSpec((1, HID), lambda i: (0, 0)),
            pl.BlockSpec((1, HID), lambda i: (0, 0)),
        ],
        out_specs=pl.BlockSpec((bn, HID), lambda i: (i, 0)),
        out_shape=jax.ShapeDtypeStruct((n, HID), jnp.float32),
    )(num, den, b.reshape(1, -1), g.reshape(1, -1), beta.reshape(1, -1))


def _gcn_post_block(agg_ref, b_ref, g_ref, beta_ref, o_ref):
    o_ref[...] = _ln_relu(agg_ref[...] + b_ref[...], g_ref[...], beta_ref[...])


def _gcn_post(agg, b, g, beta, bn=1000):
    n = agg.shape[0]
    return pl.pallas_call(
        _gcn_post_block,
        grid=(n // bn,),
        in_specs=[
            pl.BlockSpec((bn, HID), lambda i: (i, 0)),
            pl.BlockSpec((1, HID), lambda i: (0, 0)),
            pl.BlockSpec((1, HID), lambda i: (0, 0)),
            pl.BlockSpec((1, HID), lambda i: (0, 0)),
        ],
        out_specs=pl.BlockSpec((bn, HID), lambda i: (i, 0)),
        out_shape=jax.ShapeDtypeStruct((n, HID), jnp.float32),
    )(agg, b.reshape(1, -1), g.reshape(1, -1), beta.reshape(1, -1))


def _edge_scores_block(ea_ref, we_ref, att_ref, o_ref):
    ew = jnp.dot(ea_ref[...], we_ref[...], preferred_element_type=jnp.float32)
    e3 = ew.reshape(ew.shape[0], HEADS, HC)
    o_ref[...] = jnp.sum(e3 * att_ref[...], axis=-1)


def _edge_scores(ea, We, att_e, bn=8000):
    e = ea.shape[0]
    return pl.pallas_call(
        _edge_scores_block,
        grid=(e // bn,),
        in_specs=[
            pl.BlockSpec((bn, 16), lambda i: (i, 0)),
            pl.BlockSpec((16, HID), lambda i: (0, 0)),
            pl.BlockSpec((1, HEADS, HC), lambda i: (0, 0, 0)),
        ],
        out_specs=pl.BlockSpec((bn, HEADS), lambda i: (i, 0)),
        out_shape=jax.ShapeDtypeStruct((e, HEADS), jnp.float32),
    )(ea, We, att_e)


def _pool_head_block(h_ref, w0, b0, g0, be0, w1, b1, g1, be1, w2, b2,
                     o_ref, ssum, smax):
    i = pl.program_id(0)
    blk = h_ref[...]
    bs = jnp.sum(blk, axis=0, keepdims=True)
    bm = jnp.max(blk, axis=0, keepdims=True)

    @pl.when(i == 0)
    def _():
        ssum[...] = bs
        smax[...] = bm

    @pl.when(i > 0)
    def _():
        ssum[...] = ssum[...] + bs
        smax[...] = jnp.maximum(smax[...], bm)

    @pl.when(i == pl.num_programs(0) - 1)
    def _():
        add_p = ssum[...]
        pooled = jnp.concatenate([add_p / N, smax[...], add_p], axis=1)
        hi = lax.Precision.HIGHEST
        z = jnp.dot(pooled, w0[...], preferred_element_type=jnp.float32,
                    precision=hi)
        z = z + b0[...]
        m = jnp.mean(z, axis=-1, keepdims=True)
        v = jnp.mean((z - m) ** 2, axis=-1, keepdims=True)
        z = jnp.maximum((z - m) * lax.rsqrt(v + 1e-5) * g0[...] + be0[...], 0.)
        z = jnp.dot(z, w1[...], preferred_element_type=jnp.float32,
                    precision=hi) + b1[...]
        m = jnp.mean(z, axis=-1, keepdims=True)
        v = jnp.mean((z - m) ** 2, axis=-1, keepdims=True)
        z = jnp.maximum((z - m) * lax.rsqrt(v + 1e-5) * g1[...] + be1[...], 0.)
        o_ref[...] = jnp.dot(z, w2[...], preferred_element_type=jnp.float32,
                             precision=hi) + b2[...]


def _pool_head(h, p, bn=1000):
    n = h.shape[0]
    c = lambda i: (0, 0)
    return pl.pallas_call(
        _pool_head_block,
        grid=(n // bn,),
        in_specs=[
            pl.BlockSpec((bn, HID), lambda i: (i, 0)),
            pl.BlockSpec((3 * HID, HID), c),
            pl.BlockSpec((1, HID), c),
            pl.BlockSpec((1, HID), c),
            pl.BlockSpec((1, HID), c),
            pl.BlockSpec((HID, HID), c),
            pl.BlockSpec((1, HID), c),
            pl.BlockSpec((1, HID), c),
            pl.BlockSpec((1, HID), c),
            pl.BlockSpec((HID, 1), c),
            pl.BlockSpec((1, 1), c),
        ],
        out_specs=pl.BlockSpec((1, 1), c),
        out_shape=jax.ShapeDtypeStruct((1, 1), jnp.float32),
        scratch_shapes=[
            pltpu.VMEM((1, HID), jnp.float32),
            pltpu.VMEM((1, HID), jnp.float32),
        ],
    )(h, p["m0_W"], p["m0_b"].reshape(1, -1), p["m0_g"].reshape(1, -1),
      p["m0_beta"].reshape(1, -1), p["m1_W"], p["m1_b"].reshape(1, -1),
      p["m1_g"].reshape(1, -1), p["m1_beta"].reshape(1, -1), p["m2_W"],
      p["m2_b"].reshape(1, -1))


_MESH = dict(core_axis_name="c", subcore_axis_name="s")


def _sc_deg_body(d_hbm, zinit, out, acc, dbuf, stage):
    c = lax.axis_index("c")
    t = lax.axis_index("s")
    w = t * 2 + c
    pltpu.sync_copy(zinit.at[c, pl.ds(t * NROWS, NROWS)],
                    acc.at[pl.ds(t * NROWS, NROWS)])
    lanes = jnp.arange(16, dtype=jnp.int32)
    onev = jnp.where(lanes == 0, 1.0, 0.0).astype(jnp.float32)

    def fill(r, carry):
        stage[r, pl.ds(0, 16)] = onev
        return carry
    lax.fori_loop(0, 2000, fill, None)
    plsc.subcore_barrier()

    def chunk(i, carry):
        b = w * EW32 + i * 2000
        pltpu.sync_copy(d_hbm.at[pl.ds(b, 2000)], dbuf)
        pltpu.sync_copy(stage, acc.at[dbuf], add=True)
        return carry
    lax.fori_loop(0, EW32 // 2000, chunk, None)
    plsc.subcore_barrier()
    pltpu.sync_copy(acc.at[pl.ds(t * NROWS, NROWS)],
                    out.at[c, pl.ds(t * NROWS, NROWS)])


def _sc_deg(d):
    zinit = jnp.zeros((2, NP, 16), jnp.float32)
    k = pl.kernel(
        _sc_deg_body,
        out_type=jax.ShapeDtypeStruct((2, NP, 16), jnp.float32),
        mesh=plsc.VectorSubcoreMesh(**_MESH),
        compiler_params=pltpu.CompilerParams(use_tc_tiling_on_sc=False, needs_layout_passes=False),
        scratch_types=[
            pltpu.VMEM_SHARED((NP, 16), jnp.float32),
            pltpu.VMEM((2000,), jnp.int32),
            pltpu.VMEM((2000, 16), jnp.float32),
        ],
    )
    return k(d, zinit)


def _sc_norm_body(s_hbm, d_hbm, dinv_hbm, out, dt, sbuf, dbuf, nbuf):
    c = lax.axis_index("c")
    t = lax.axis_index("s")
    w = t * 2 + c
    pltpu.sync_copy(dinv_hbm, dt)
    lanes = jnp.arange(16, dtype=jnp.int32)

    def chunk(i, carry):
        b = w * EW32 + i * 2000
        pltpu.sync_copy(s_hbm.at[pl.ds(b, 2000)], sbuf)
        pltpu.sync_copy(d_hbm.at[pl.ds(b, 2000)], dbuf)

        @plsc.parallel_loop(0, 125, unroll=4)
        def grp(g):
            s16 = sbuf[pl.ds(g * 16, 16)]
            d16 = dbuf[pl.ds(g * 16, 16)]
            nv = plsc.load_gather(dt, [s16]) * plsc.load_gather(dt, [d16])
            nbuf[pl.ds(g * 16, 16)] = nv
        pltpu.sync_copy(nbuf, out.at[pl.ds(b, 2000)])
        return carry
    lax.fori_loop(0, EW32 // 2000, chunk, None)


def _sc_norm(s, d, dinv):
    k = pl.kernel(
        _sc_norm_body,
        out_type=jax.ShapeDtypeStruct((E,), jnp.float32),
        mesh=plsc.VectorSubcoreMesh(**_MESH),
        compiler_params=pltpu.CompilerParams(use_tc_tiling_on_sc=False, needs_layout_passes=False),
        scratch_types=[
            pltpu.VMEM((N,), jnp.float32),
            pltpu.VMEM((2000,), jnp.int32),
            pltpu.VMEM((2000,), jnp.int32),
            pltpu.VMEM((2000,), jnp.float32),
        ],
    )
    return k(s, d, dinv)


def _sc_gat_body(xt, sth, dth, aeh, s2, d2, d_hbm, inith, out,
                 acc, xbuf, sgbuf, dgbuf, sbuf, d2buf, dbuf, aebuf, pbuf,
                 sem):
    c = lax.axis_index("c")
    t = lax.axis_index("s")
    pltpu.sync_copy(inith.at[c, pl.ds(t * NROWS, NROWS)],
                    acc.at[pl.ds(t * NROWS, NROWS)])
    pbuf[pl.ds(4 * KA, 16)] = jnp.zeros((16,), jnp.float32)
    plsc.subcore_barrier()
    lanes = jnp.arange(16, dtype=jnp.int32)
    base = t * EW

    def chunk(i, carry):
        b = base + i * KA
        cb = c * E + b
        pltpu.sync_copy(s2.at[pl.ds(cb, KA)], sbuf)
        pltpu.sync_copy(d2.at[pl.ds(cb, KA)], d2buf)
        pltpu.sync_copy(d_hbm.at[pl.ds(b, KA)], dbuf)
        for h in range(4):
            pltpu.sync_copy(aeh.at[pl.ds((c * 4 + h) * E + b, KA)],
                            aebuf.at[pl.ds(h * KA, KA)])
        g1 = pltpu.async_copy(xt.at[sbuf], xbuf, sem)
        g2 = pltpu.async_copy(sth.at[sbuf], sgbuf, sem)
        g3 = pltpu.async_copy(dth.at[d2buf], dgbuf, sem)
        g1.wait()
        g2.wait()
        g3.wait()

        @plsc.parallel_loop(0, KA // 16, unroll=2)
        def score(g):
            lid = g * 16 + lanes
            for h in range(4):
                hv = jnp.full((16,), h, jnp.int32)
                a_s = plsc.load_gather(sgbuf, [lid, hv])
                a_d = plsc.load_gather(dgbuf, [lid, hv])
                raw = a_s + a_d + aebuf[pl.ds(h * KA + g * 16, 16)]
                raw = jnp.where(raw >= 0, raw, 0.2 * raw)
                pbuf[pl.ds(h * KA + g * 16, 16)] = jnp.exp(raw)

        @plsc.parallel_loop(0, KA, unroll=4)
        def msg(e):
            pv = plsc.load_gather(
                pbuf, [jnp.where(lanes < 4, lanes * KA + e, 4 * KA)])
            for h in range(4):
                wv = plsc.load_gather(
                    pbuf, [jnp.full((16,), h * KA, jnp.int32) + e])
                for q in range(2):
                    col = h * 32 + q * 16
                    xbuf[e, pl.ds(col, 16)] = xbuf[e, pl.ds(col, 16)] * wv
            xbuf[e, pl.ds(128, 16)] = pv
        pltpu.sync_copy(xbuf, acc.at[dbuf], add=True)
        return carry
    lax.fori_loop(0, EW // KA, chunk, None)
    plsc.subcore_barrier()
    pltpu.sync_copy(acc.at[pl.ds(t * NROWS, NROWS)],
                    out.at[c, pl.ds(t * NROWS, NROWS)])


def _sc_gat(xt, sth, dth, aeh, s2, d2, d, inith):
    k = pl.kernel(
        _sc_gat_body,
        out_type=jax.ShapeDtypeStruct((2, NP, ROWW), jnp.float32),
        mesh=plsc.VectorSubcoreMesh(**_MESH),
        compiler_params=pltpu.CompilerParams(use_tc_tiling_on_sc=False, needs_layout_passes=False),
        scratch_types=[
            pltpu.VMEM_SHARED((NP, ROWW), jnp.float32),
            pltpu.VMEM((KA, ROWW), jnp.float32),
            pltpu.VMEM((KA, 8), jnp.float32),
            pltpu.VMEM((KA, 8), jnp.float32),
            pltpu.VMEM((KA,), jnp.int32),
            pltpu.VMEM((KA,), jnp.int32),
            pltpu.VMEM((KA,), jnp.int32),
            pltpu.VMEM((4 * KA,), jnp.float32),
            pltpu.VMEM((4 * KA + 16,), jnp.float32),
            pltpu.SemaphoreType.DMA,
        ],
    )
    return k(xt, sth, dth, aeh, s2, d2, d, inith)


def _sc_gcn_body(xt, s2, d_hbm, nrm, inith, out,
                 acc, xbuf, sbuf, dbuf, nbuf, sem):
    c = lax.axis_index("c")
    t = lax.axis_index("s")
    pltpu.sync_copy(inith.at[c, pl.ds(t * NROWS, NROWS)],
                    acc.at[pl.ds(t * NROWS, NROWS)])
    plsc.subcore_barrier()
    base = t * EW

    def chunk(i, carry):
        b = base + i * KC
        pltpu.sync_copy(s2.at[pl.ds(c * E + b, KC)], sbuf)
        pltpu.sync_copy(d_hbm.at[pl.ds(b, KC)], dbuf)
        pltpu.sync_copy(nrm.at[pl.ds(b, KC)], nbuf)
        pltpu.async_copy(xt.at[sbuf], xbuf, sem).wait()

        @plsc.parallel_loop(0, KC, unroll=4)
        def msg(e):
            wv = plsc.load_gather(nbuf, [jnp.full((16,), 0, jnp.int32) + e])
            for q in range(8):
                xbuf[e, pl.ds(q * 16, 16)] = xbuf[e, pl.ds(q * 16, 16)] * wv
        pltpu.sync_copy(xbuf, acc.at[dbuf], add=True)
        return carry
    lax.fori_loop(0, EW // KC, chunk, None)
    plsc.subcore_barrier()
    pltpu.sync_copy(acc.at[pl.ds(t * NROWS, NROWS)],
                    out.at[c, pl.ds(t * NROWS, NROWS)])


def _sc_gcn(xt, s2, d, nrm, inith):
    k = pl.kernel(
        _sc_gcn_body,
        out_type=jax.ShapeDtypeStruct((2, NP, 128), jnp.float32),
        mesh=plsc.VectorSubcoreMesh(**_MESH),
        compiler_params=pltpu.CompilerParams(use_tc_tiling_on_sc=False, needs_layout_passes=False),
        scratch_types=[
            pltpu.VMEM_SHARED((NP, 128), jnp.float32),
            pltpu.VMEM((KC, 128), jnp.float32),
            pltpu.VMEM((KC,), jnp.int32),
            pltpu.VMEM((KC,), jnp.int32),
            pltpu.VMEM((KC,), jnp.float32),
            pltpu.SemaphoreType.DMA,
        ],
    )
    return k(xt, s2, d, nrm, inith)


def _padn(a):
    return jnp.concatenate(
        [a, jnp.zeros((2, NP - N, a.shape[2]), jnp.float32)], axis=1)


def kernel(x, edge_index, edge_attr, batch, params):
    f32 = jnp.float32
    src = edge_index[0]
    dst = edge_index[1]
    s2 = jnp.concatenate([src, src + N])
    d2 = jnp.concatenate([dst, dst + N])

    h = _in_dense(x, params["in_W"], params["in_b"], params["in_g"],
                  params["in_beta"])

    degp = _sc_deg(dst)
    deg = 1.0 + degp[0, :N, 0] + degp[1, :N, 0]
    dinv = deg ** -0.5
    nrm = _sc_norm(src, dst, dinv)
    dinv2 = dinv * dinv

    mean_ea = jnp.mean(edge_attr, axis=0)

    for i in range(5):
        p = params["layers"][i]
        if i % 2 == 0:
            xl, a_s, a_d = _gat_mm(h, p["W"], p["att_src"], p["att_dst"])
            ae = _edge_scores(edge_attr, p["W_e"], p["att_e"])
            ae_loop = jnp.sum(
                (mean_ea @ p["W_e"]).reshape(HEADS, HC) * p["att_e"][0],
                axis=-1)

            p_self = jnp.exp(jnp.where(
                a_s + a_d + ae_loop[None] >= 0,
                a_s + a_d + ae_loop[None],
                0.2 * (a_s + a_d + ae_loop[None])))

            xl_sp = xl.reshape(N, 2, 128).transpose(1, 0, 2)
            xt = jnp.concatenate(
                [xl_sp, jnp.zeros((2, N, 16), f32)], axis=2
            ).reshape(2 * N, ROWW)
            z4 = jnp.zeros((2, N, 4), f32)
            sth = jnp.concatenate(
                [a_s.reshape(N, 2, 4).transpose(1, 0, 2), z4],
                axis=2).reshape(2 * N, 8)
            dth = jnp.concatenate(
                [a_d.reshape(N, 2, 4).transpose(1, 0, 2), z4],
                axis=2).reshape(2 * N, 8)
            aeh = ae.T.reshape(-1)
            ps_sp = p_self.reshape(N, 2, 4).transpose(1, 0, 2)
            inith = jnp.concatenate(
                [xl_sp * jnp.repeat(ps_sp, HC, axis=2),
                 ps_sp, jnp.zeros((2, N, 12), f32)], axis=2)

            agg = _sc_gat(xt, sth, dth, aeh, s2, d2, dst, _padn(inith))
            num = jnp.concatenate([agg[0, :N, :128], agg[1, :N, :128]], axis=1)
            den = jnp.concatenate(
                [agg[0, :N, 128:132], agg[1, :N, 128:132]], axis=1)
            h = _gat_post(num, den, p["b"], params["ln_g"][i],
                          params["ln_b"][i])
        else:
            xw = _plain_mm(h, p["W"])
            xw_sp = xw.reshape(N, 2, 128).transpose(1, 0, 2)
            inith = xw_sp * dinv2[None, :, None]
            agg = _sc_gcn(xw_sp.reshape(2 * N, 128), s2, dst, nrm,
                          _padn(inith))
            aggf = jnp.concatenate([agg[0, :N], agg[1, :N]], axis=1)
            h = _gcn_post(aggf, p["b"], params["ln_g"][i],
                          params["ln_b"][i])

    return _pool_head(h, params)
